# Initial kernel scaffold; baseline (speedup 1.0000x reference)
import jax
import jax.numpy as jnp
from jax.experimental import pallas as pl


def kernel(edge_index, x, edge_attr, node_emb_W, node_emb_b, edge_emb_W, edge_emb_b, f_W1, f_b1, f_W2, f_b2, g_W1, g_b1, g_W2, g_b2, gru_Wih, gru_bih, gru_Whh, gru_bhh, alpha_W1, alpha_b1, alpha_W2, alpha_b2, npred_W1, npred_b1, npred_W2, npred_b2, epred_W1, epred_b1, epred_W2, epred_b2):
    raise NotImplementedError("write your pallas kernel here")


# speedup vs baseline: 1.8745x; 1.8745x over previous
import functools

import jax
import jax.numpy as jnp
from jax import lax
from jax.experimental import pallas as pl
from jax.experimental.pallas import tpu as pltpu

N_NODES = 10000
N_EDGES = 320000
HID = 64
K = 20

_DOT = functools.partial(jnp.dot, preferred_element_type=jnp.float32)


def _embed_body(x_ref, wt_ref, b_ref, out_ref):
    out_ref[...] = _DOT(x_ref[...], wt_ref[...]) + b_ref[...]


def _embed(x, node_emb_W, node_emb_b):
    return pl.pallas_call(
        _embed_body,
        out_shape=jax.ShapeDtypeStruct((N_NODES, HID), jnp.float32),
    )(x, node_emb_W.T, node_emb_b.reshape(1, HID))


def _ttrans_body(hv_ref, a_ref, out_ref):
    out_ref[...] = _DOT(hv_ref[...], a_ref[...])


def _ttrans(h_v, A):
    return pl.pallas_call(
        _ttrans_body,
        out_shape=jax.ShapeDtypeStruct((N_NODES, 2 * HID), jnp.float32),
    )(h_v, A)


_EB = 2560


def _mlp_body(pre_ref, ea_ref, uc_ref, w2f_ref, w2g_ref, out_ref):
    pre = pre_ref[...]
    ea = ea_ref[...]
    uc = uc_ref[...]
    hf = jnp.maximum(pre[:, :HID] + ea * uc[0:1, :] + uc[1:2, :], 0.0)
    hg = jnp.maximum(pre[:, HID:] + ea * uc[2:3, :] + uc[3:4, :], 0.0)
    m = _DOT(hf, w2f_ref[...]) + uc[4:5, :]
    a = _DOT(hg, w2g_ref[...]) + uc[5:6, :]
    out_ref[...] = m * a


def _edge_mlp(pre, ea_col, uc, w2fT, w2gT):
    grid = N_EDGES // _EB
    return pl.pallas_call(
        _mlp_body,
        grid=(grid,),
        in_specs=[
            pl.BlockSpec((_EB, 2 * HID), lambda i: (i, 0)),
            pl.BlockSpec((_EB, 1), lambda i: (i, 0)),
            pl.BlockSpec((8, HID), lambda i: (0, 0)),
            pl.BlockSpec((HID, HID), lambda i: (0, 0)),
            pl.BlockSpec((HID, HID), lambda i: (0, 0)),
        ],
        out_specs=pl.BlockSpec((_EB, HID), lambda i: (i, 0)),
        out_shape=jax.ShapeDtypeStruct((N_EDGES, HID), jnp.float32),
    )(pre, ea_col, uc, w2fT, w2gT)


def _gi_body(hv_ref, agg_ref, wa_ref, wb_ref, b_ref, out_ref):
    out_ref[...] = (_DOT(hv_ref[...], wa_ref[...])
                    + _DOT(agg_ref[...], wb_ref[...]) + b_ref[...])


def _gi(h_v, agg, WihT_a, WihT_b, bias):
    return pl.pallas_call(
        _gi_body,
        out_shape=jax.ShapeDtypeStruct((N_NODES, 3 * HID), jnp.float32),
    )(h_v, agg, WihT_a, WihT_b, bias.reshape(1, 3 * HID))


def _scan_body(gi_ref, whhT_ref, bhn_ref, out_ref, hsum_ref):
    whhT = whhT_ref[...]
    bhn = bhn_ref[...]

    def step(t, carry):
        h, hs = carry
        gh = _DOT(h, whhT)
        grow = gi_ref[pl.ds(t, 1), :]
        r = jax.nn.sigmoid(grow[:, :HID] + gh[:, :HID])
        z = jax.nn.sigmoid(grow[:, HID:2 * HID] + gh[:, HID:2 * HID])
        n = jnp.tanh(grow[:, 2 * HID:] + r * (gh[:, 2 * HID:] + bhn))
        h = (1.0 - z) * n + z * h
        out_ref[pl.ds(t, 1), :] = h
        return h, hs + h

    h0 = jnp.zeros((1, HID), jnp.float32)
    _, hs = lax.fori_loop(0, N_NODES, step, (h0, h0))
    hsum_ref[...] = hs


def _gru_scan(GI, WhhT, bhh_n):
    return pl.pallas_call(
        _scan_body,
        out_shape=(jax.ShapeDtypeStruct((N_NODES, HID), jnp.float32),
                   jax.ShapeDtypeStruct((1, HID), jnp.float32)),
    )(GI, WhhT, bhh_n.reshape(1, HID))


_NB = 2000


def _headsB_body(hv_ref, ge_ref, hl_ref, npw_ref, npb_ref, alw_ref, alb_ref,
                 np2_ref, al2_ref, npsum_ref, alsum_ref):
    i = pl.program_id(0)
    ge = ge_ref[...]
    hl = hl_ref[...]
    hv = hv_ref[...]
    h1 = jnp.maximum(_DOT(ge, npw_ref[:HID, :]) + _DOT(hv, npw_ref[HID:, :])
                     + npb_ref[...], 0.0)
    npo = _DOT(h1, np2_ref[...])
    a1 = jnp.maximum(_DOT(ge, alw_ref[:HID, :]) + _DOT(hl, alw_ref[HID:2 * HID, :])
                     + _DOT(hv, alw_ref[2 * HID:, :]) + alb_ref[...], 0.0)
    alo = _DOT(a1, al2_ref[...])

    @pl.when(i == 0)
    def _():
        npsum_ref[...] = jnp.zeros_like(npsum_ref)
        alsum_ref[...] = jnp.zeros_like(alsum_ref)

    npsum_ref[...] += jnp.sum(npo, axis=0, keepdims=True)
    alsum_ref[...] += jnp.sum(alo, axis=0, keepdims=True)


def _headsB(h_v, ge, h_last, npW1T, npb1, alW1T, alb1, npW2T, alW2T):
    grid = N_NODES // _NB
    return pl.pallas_call(
        _headsB_body,
        grid=(grid,),
        in_specs=[
            pl.BlockSpec((_NB, HID), lambda i: (i, 0)),
            pl.BlockSpec((1, HID), lambda i: (0, 0)),
            pl.BlockSpec((1, HID), lambda i: (0, 0)),
            pl.BlockSpec((2 * HID, HID), lambda i: (0, 0)),
            pl.BlockSpec((1, HID), lambda i: (0, 0)),
            pl.BlockSpec((3 * HID, HID), lambda i: (0, 0)),
            pl.BlockSpec((1, HID), lambda i: (0, 0)),
            pl.BlockSpec((HID, 16), lambda i: (0, 0)),
            pl.BlockSpec((HID, K), lambda i: (0, 0)),
        ],
        out_specs=(pl.BlockSpec((1, 16), lambda i: (0, 0)),
                   pl.BlockSpec((1, K), lambda i: (0, 0))),
        out_shape=(jax.ShapeDtypeStruct((1, 16), jnp.float32),
                   jax.ShapeDtypeStruct((1, K), jnp.float32)),
    )(h_v, ge, h_last, npW1T, npb1.reshape(1, HID), alW1T,
      alb1.reshape(1, HID), npW2T, alW2T)


def _headsC_body(hv_ref, nps_ref, als_ref, e1_ref, e1b_ref, e2_ref, e2b_ref,
                 npb2_ref, pv_ref, pe_ref):
    i = pl.program_id(0)
    al = als_ref[...]
    al = al - jnp.max(al, axis=1, keepdims=True)
    ale = jnp.exp(al)
    alphas = ale / jnp.sum(ale, axis=1, keepdims=True)

    @pl.when(i == 0)
    def _():
        npv = nps_ref[...] / float(N_NODES) + npb2_ref[...]
        npv = npv - jnp.max(npv, axis=1, keepdims=True)
        e = jnp.exp(npv)
        pv_ref[...] = e / jnp.sum(e, axis=1, keepdims=True)

    hv = hv_ref[...]
    h1 = jnp.maximum(_DOT(hv, e1_ref[...]) + e1b_ref[...], 0.0)
    lt = _DOT(h1, e2_ref[...]) + e2b_ref[...]
    groups = [lt[:, j * K:(j + 1) * K] for j in range(5)]
    mx = groups[0]
    for g in groups[1:]:
        mx = jnp.maximum(mx, g)
    es = [jnp.exp(g - mx) for g in groups]
    den = es[0]
    for e in es[1:]:
        den = den + e
    cols = [jnp.sum(alphas * (e / den), axis=1, keepdims=True) for e in es]
    pe_ref[...] = jnp.concatenate(cols, axis=1)


def _headsC(h_v, npsum, alsum, eW1T, eb1, eW2T, eb2, npb2):
    grid = N_NODES // _NB
    return pl.pallas_call(
        _headsC_body,
        grid=(grid,),
        in_specs=[
            pl.BlockSpec((_NB, HID), lambda i: (i, 0)),
            pl.BlockSpec((1, 16), lambda i: (0, 0)),
            pl.BlockSpec((1, K), lambda i: (0, 0)),
            pl.BlockSpec((HID, HID), lambda i: (0, 0)),
            pl.BlockSpec((1, HID), lambda i: (0, 0)),
            pl.BlockSpec((HID, 5 * K), lambda i: (0, 0)),
            pl.BlockSpec((1, 5 * K), lambda i: (0, 0)),
            pl.BlockSpec((1, 16), lambda i: (0, 0)),
        ],
        out_specs=(pl.BlockSpec((1, 16), lambda i: (0, 0)),
                   pl.BlockSpec((_NB, 5), lambda i: (i, 0))),
        out_shape=(jax.ShapeDtypeStruct((1, 16), jnp.float32),
                   jax.ShapeDtypeStruct((N_NODES, 5), jnp.float32)),
    )(h_v, npsum, alsum, eW1T, eb1.reshape(1, HID), eW2T,
      eb2.reshape(1, 5 * K), npb2.reshape(1, 16))


def kernel(edge_index, x, edge_attr,
           node_emb_W, node_emb_b, edge_emb_W, edge_emb_b,
           f_W1, f_b1, f_W2, f_b2,
           g_W1, g_b1, g_W2, g_b2,
           gru_Wih, gru_bih, gru_Whh, gru_bhh,
           alpha_W1, alpha_b1, alpha_W2, alpha_b2,
           npred_W1, npred_b1, npred_W2, npred_b2,
           epred_W1, epred_b1, epred_W2, epred_b2):
    x = x.astype(jnp.float32)
    edge_attr = edge_attr.astype(jnp.float32)
    src = edge_index[0]
    dst = edge_index[1]

    h_v = _embed(x, node_emb_W, node_emb_b)

    we = edge_emb_W[:, 0]
    ea_col = edge_attr.reshape(N_EDGES, 1)

    for l in range(3):
        fW1, gW1 = f_W1[l], g_W1[l]
        A_dst = jnp.concatenate([fW1[:, :HID].T, gW1[:, :HID].T], axis=1)
        A_src = jnp.concatenate([fW1[:, HID:2 * HID].T, gW1[:, HID:2 * HID].T], axis=1)
        T_dst = _ttrans(h_v, A_dst)
        T_src = _ttrans(h_v, A_src)
        u_f = fW1[:, 2 * HID:] @ we
        c_f = fW1[:, 2 * HID:] @ edge_emb_b + f_b1[l]
        u_g = gW1[:, 2 * HID:] @ we
        c_g = gW1[:, 2 * HID:] @ edge_emb_b + g_b1[l]
        uc = jnp.stack([u_f, c_f, u_g, c_g, f_b2[l], g_b2[l],
                        jnp.zeros_like(u_f), jnp.zeros_like(u_f)], axis=0)

        pre = T_dst[dst] + T_src[src]

        prod = _edge_mlp(pre, ea_col, uc, f_W2[l].T, g_W2[l].T)

        agg = jax.ops.segment_sum(prod, dst, num_segments=N_NODES)

        bias = gru_bih[l] + jnp.concatenate(
            [gru_bhh[l][:2 * HID], jnp.zeros((HID,), jnp.float32)])
        GI = _gi(h_v, agg, gru_Wih[l][:, :HID].T, gru_Wih[l][:, HID:].T, bias)
        h_v, hsum = _gru_scan(GI, gru_Whh[l].T, gru_bhh[l][2 * HID:])

    ge = hsum / float(N_NODES)
    h_last = h_v[N_NODES - 1:N_NODES, :]

    npsum, alsum = _headsB(h_v, ge, h_last, npred_W1.T, npred_b1,
                           alpha_W1.T, alpha_b1, npred_W2.T, alpha_W2.T)
    alsum = alsum + alpha_b2.reshape(1, K) * float(N_NODES)
    pv, pe = _headsC(h_v, npsum, alsum, epred_W1.T, epred_b1, epred_W2.T,
                     epred_b2, npred_b2)
    return (pv.reshape(16), pe)
